# baseline (speedup 1.0000x reference)
import jax
import jax.numpy as jnp
from jax.experimental import pallas as pl
from jax.experimental.pallas import tpu as pltpu

B, N, D = 1, 2048, 768
H, KVH, DH = 12, 12, 64
BLK, STRIDE = 16, 8
SELBLK, NSEL = 16, 4
WIN = 64
SCALE = DH ** -0.5
NCB = (N - BLK) // STRIDE + 1
NSB = N // SELBLK
TQ = 256
BF16 = jnp.bfloat16
F32 = jnp.float32


def _bdot(a, b, dims=None):
    if dims is None:
        dims = (((a.ndim - 1,), (0,)), ((), ()))
    return jax.lax.dot_general(a.astype(BF16), b.astype(BF16), dims,
                               preferred_element_type=F32)


def _proj_kernel(x_ref, g_ref, wq_ref, wk_ref, wv_ref, wc_ref, bc_ref,
                 qb_ref, kb_ref, vb_ref, k_ref, v_ref, gc_ref):
    x = x_ref[:]
    xn = x * jax.lax.rsqrt(jnp.mean(x * x, axis=-1, keepdims=True) + 1e-6)
    xn = xn * g_ref[:]
    xnb = xn.astype(BF16)
    dd = lambda w: jax.lax.dot_general(xnb, w, (((1,), (0,)), ((), ())),
                                       preferred_element_type=F32)
    q = dd(wq_ref[:])
    k = dd(wk_ref[:])
    v = dd(wv_ref[:])
    qb_ref[:] = q.astype(BF16)
    kb_ref[:] = k.astype(BF16)
    vb_ref[:] = v.astype(BF16)
    k_ref[:] = k
    v_ref[:] = v
    gc_ref[:] = jax.nn.sigmoid(dd(wc_ref[:]) + bc_ref[:])


def _compress_kernel(kr_ref, vr_ref, kpe_ref, vpe_ref, wck_ref, wcv_ref,
                     mem_ref, ckf_ref, cvf_ref):
    k8 = kr_ref[0]
    v8 = vr_ref[0]
    half = STRIDE * DH
    kpe = kpe_ref[0]
    vpe = vpe_ref[0]
    k8s = jnp.concatenate([k8[1:], k8[:1]], axis=0)
    v8s = jnp.concatenate([v8[1:], v8[:1]], axis=0)
    kbf = jnp.concatenate([k8 + kpe[:, :half], k8s + kpe[:, half:]], axis=1)
    vbf = jnp.concatenate([v8 + vpe[:, :half], v8s + vpe[:, half:]], axis=1)
    ck = _bdot(kbf, wck_ref[0])
    cv = _bdot(vbf, wcv_ref[0])
    ckf_ref[0] = jnp.concatenate([mem_ref[0, 0], ck[:NCB]], axis=0)
    cvf_ref[0] = jnp.concatenate([mem_ref[1, 0], cv[:NCB]], axis=0)


def _attn_kernel(q_ref, k_ref, v_ref, ckf_ref, cvf_ref,
                 co_ref, fo_ref, so_ref, scr_ref):
    i = pl.program_id(1)
    qb = q_ref[0]
    ckf = ckf_ref[0]
    cvf = cvf_ref[0]
    t = i * TQ + jax.lax.broadcasted_iota(jnp.int32, (TQ, 1), 0)

    cs = jax.lax.dot_general(qb, ckf.astype(BF16), (((1,), (1,)), ((), ())),
                             preferred_element_type=F32) * SCALE
    jc = jax.lax.broadcasted_iota(jnp.int32, (TQ, NCB + 1), 1)
    cmask = (jc == 0) | ((jc - 1) * STRIDE + BLK - 1 <= t)
    cs = jnp.where(cmask, cs, -1e30)
    cm = jnp.max(cs, axis=-1, keepdims=True)
    ce = jnp.exp(cs - cm)
    cp = ce / jnp.sum(ce, axis=-1, keepdims=True)
    co_ref[0] = _bdot(cp, cvf)

    cpr = jnp.concatenate([cp[:, 1:], cp[:, :1]], axis=1)
    impw = cp + cpr
    lane = jax.lax.broadcasted_iota(jnp.int32, (TQ, NCB + 1), 1)
    odd = (lane % 2) == 1
    impw = jnp.where(odd, impw, -1.0)
    impw = jnp.where(lane == NCB, cp[:, NCB:NCB + 1], impw)
    sels = []
    for _ in range(NSEL):
        idx = jnp.argmax(impw, axis=-1, keepdims=True).astype(jnp.int32)
        sels.append(jax.lax.shift_right_logical(idx, 1))
        impw = jnp.where(lane == idx, -2.0, impw)

    tb = t // SELBLK

    def score_tile(j, m0):
        kt = k_ref[0, pl.ds(j * TQ, TQ), :]
        st = jax.lax.dot_general(qb, kt, (((1,), (1,)), ((), ())),
                                 preferred_element_type=F32) * SCALE
        jcol = j * TQ + jax.lax.broadcasted_iota(jnp.int32, (TQ, TQ), 1)
        jb = jcol // SELBLK
        fmask = (jb == sels[0]) | (jb == sels[1]) | (jb == sels[2]) \
            | (jb == sels[3]) | (jb == tb)
        fmask = fmask & (jcol <= t)
        st = jnp.where(fmask, st, -1e30)
        scr_ref[:, pl.ds(j * TQ, TQ)] = st
        return jnp.maximum(m0, jnp.max(st, axis=-1, keepdims=True))

    fm = jax.lax.fori_loop(0, i + 1, score_tile,
                           jnp.full((TQ, 1), -1e30, F32))

    def exp_tile(j, l0):
        e = jnp.exp(scr_ref[:, pl.ds(j * TQ, TQ)] - fm)
        scr_ref[:, pl.ds(j * TQ, TQ)] = e
        return l0 + jnp.sum(e, axis=-1, keepdims=True)

    fl = jax.lax.fori_loop(0, i + 1, exp_tile, jnp.zeros((TQ, 1), F32))

    def pv_tile(j, acc0):
        fp = (scr_ref[:, pl.ds(j * TQ, TQ)] / fl).astype(BF16)
        vt = v_ref[0, pl.ds(j * TQ, TQ), :]
        return acc0 + jax.lax.dot_general(fp, vt, (((1,), (0,)), ((), ())),
                                          preferred_element_type=F32)

    fo_ref[0] = jax.lax.fori_loop(0, i + 1, pv_tile, jnp.zeros((TQ, DH), F32))

    start = jnp.maximum(i - 1, 0) * TQ
    kw = k_ref[0, pl.ds(start, 2 * TQ), :]
    vw = v_ref[0, pl.ds(start, 2 * TQ), :]
    sw = jax.lax.dot_general(qb, kw, (((1,), (1,)), ((), ())),
                             preferred_element_type=F32) * SCALE
    jcw = start + jax.lax.broadcasted_iota(jnp.int32, (TQ, 2 * TQ), 1)
    smask = (jcw <= t) & (t - jcw < WIN)
    ws = jnp.where(smask, sw, -1e30)
    wm = jnp.max(ws, axis=-1, keepdims=True)
    we = jnp.where(smask, jnp.exp(ws - wm), 0.0)
    wp = we / jnp.sum(we, axis=-1, keepdims=True)
    so_ref[0] = _bdot(wp, vw)


def _out_kernel(co_ref, fo_ref, so_ref, gc_ref, wo_ref, o_ref):
    gc = gc_ref[:]
    acc = jnp.zeros((TQ, D), F32)
    for h in range(H):
        comb = gc[:, h:h + 1] * co_ref[h] \
            + gc[:, H + h:H + h + 1] * fo_ref[h] \
            + gc[:, 2 * H + h:2 * H + h + 1] * so_ref[h]
        acc = acc + _bdot(comb, wo_ref[h])
    o_ref[:] = acc


def kernel(x, g_norm, Wq, Wk, Wv, k_pe, v_pe, Wck, Wcv, mem_kv,
           W_comb, b_comb, Wo):
    b, n, d = x.shape
    x2 = x.reshape(n, d)
    g2 = g_norm.reshape(1, d)
    b2 = b_comb.reshape(1, 3 * H)

    qb, kb, vb, k, v, gc = pl.pallas_call(
        _proj_kernel,
        grid=(n // TQ,),
        in_specs=[
            pl.BlockSpec((TQ, d), lambda i: (i, 0)),
            pl.BlockSpec((1, d), lambda i: (0, 0)),
            pl.BlockSpec((d, H * DH), lambda i: (0, 0)),
            pl.BlockSpec((d, KVH * DH), lambda i: (0, 0)),
            pl.BlockSpec((d, KVH * DH), lambda i: (0, 0)),
            pl.BlockSpec((d, 3 * H), lambda i: (0, 0)),
            pl.BlockSpec((1, 3 * H), lambda i: (0, 0)),
        ],
        out_specs=[
            pl.BlockSpec((TQ, H * DH), lambda i: (i, 0)),
            pl.BlockSpec((TQ, KVH * DH), lambda i: (i, 0)),
            pl.BlockSpec((TQ, KVH * DH), lambda i: (i, 0)),
            pl.BlockSpec((TQ, KVH * DH), lambda i: (i, 0)),
            pl.BlockSpec((TQ, KVH * DH), lambda i: (i, 0)),
            pl.BlockSpec((TQ, 3 * H), lambda i: (i, 0)),
        ],
        out_shape=[
            jax.ShapeDtypeStruct((n, H * DH), BF16),
            jax.ShapeDtypeStruct((n, KVH * DH), BF16),
            jax.ShapeDtypeStruct((n, KVH * DH), BF16),
            jax.ShapeDtypeStruct((n, KVH * DH), F32),
            jax.ShapeDtypeStruct((n, KVH * DH), F32),
            jax.ShapeDtypeStruct((n, 3 * H), F32),
        ],
    )(x2, g2, Wq.astype(BF16), Wk.astype(BF16), Wv.astype(BF16),
      W_comb.astype(BF16), b2)

    qh = qb.reshape(n, H, DH).transpose(1, 0, 2)
    kh = kb.reshape(n, KVH, DH).transpose(1, 0, 2)
    vh = vb.reshape(n, KVH, DH).transpose(1, 0, 2)
    kr = k.reshape(n, KVH, DH).transpose(1, 0, 2) \
        .reshape(KVH, n // STRIDE, STRIDE * DH)
    vr = v.reshape(n, KVH, DH).transpose(1, 0, 2) \
        .reshape(KVH, n // STRIDE, STRIDE * DH)
    kpe2 = k_pe.reshape(KVH, 1, BLK * DH)
    vpe2 = v_pe.reshape(KVH, 1, BLK * DH)
    mem2 = mem_kv.reshape(2, KVH, 1, DH)

    ckf, cvf = pl.pallas_call(
        _compress_kernel,
        grid=(KVH,),
        in_specs=[
            pl.BlockSpec((1, n // STRIDE, STRIDE * DH), lambda h: (h, 0, 0)),
            pl.BlockSpec((1, n // STRIDE, STRIDE * DH), lambda h: (h, 0, 0)),
            pl.BlockSpec((1, 1, BLK * DH), lambda h: (h, 0, 0)),
            pl.BlockSpec((1, 1, BLK * DH), lambda h: (h, 0, 0)),
            pl.BlockSpec((1, BLK * DH, DH), lambda h: (h, 0, 0)),
            pl.BlockSpec((1, BLK * DH, DH), lambda h: (h, 0, 0)),
            pl.BlockSpec((2, 1, 1, DH), lambda h: (0, h, 0, 0)),
        ],
        out_specs=[
            pl.BlockSpec((1, NCB + 1, DH), lambda h: (h, 0, 0)),
            pl.BlockSpec((1, NCB + 1, DH), lambda h: (h, 0, 0)),
        ],
        out_shape=[
            jax.ShapeDtypeStruct((KVH, NCB + 1, DH), F32),
            jax.ShapeDtypeStruct((KVH, NCB + 1, DH), F32),
        ],
    )(kr, vr, kpe2, vpe2, Wck.astype(BF16), Wcv.astype(BF16), mem2)

    co, fo, so = pl.pallas_call(
        _attn_kernel,
        grid=(H, n // TQ),
        in_specs=[
            pl.BlockSpec((1, TQ, DH), lambda h, i: (h, i, 0)),
            pl.BlockSpec((1, n, DH), lambda h, i: (h, 0, 0)),
            pl.BlockSpec((1, n, DH), lambda h, i: (h, 0, 0)),
            pl.BlockSpec((1, NCB + 1, DH), lambda h, i: (h, 0, 0)),
            pl.BlockSpec((1, NCB + 1, DH), lambda h, i: (h, 0, 0)),
        ],
        out_specs=[
            pl.BlockSpec((1, TQ, DH), lambda h, i: (h, i, 0)),
            pl.BlockSpec((1, TQ, DH), lambda h, i: (h, i, 0)),
            pl.BlockSpec((1, TQ, DH), lambda h, i: (h, i, 0)),
        ],
        out_shape=[
            jax.ShapeDtypeStruct((H, n, DH), F32),
            jax.ShapeDtypeStruct((H, n, DH), F32),
            jax.ShapeDtypeStruct((H, n, DH), F32),
        ],
        scratch_shapes=[pltpu.VMEM((TQ, N), F32)],
    )(qh, kh, vh, ckf, cvf)

    out = pl.pallas_call(
        _out_kernel,
        grid=(n // TQ,),
        in_specs=[
            pl.BlockSpec((H, TQ, DH), lambda i: (0, i, 0)),
            pl.BlockSpec((H, TQ, DH), lambda i: (0, i, 0)),
            pl.BlockSpec((H, TQ, DH), lambda i: (0, i, 0)),
            pl.BlockSpec((TQ, 3 * H), lambda i: (i, 0)),
            pl.BlockSpec((H, DH, d), lambda i: (0, 0, 0)),
        ],
        out_specs=pl.BlockSpec((TQ, d), lambda i: (i, 0)),
        out_shape=jax.ShapeDtypeStruct((n, d), F32),
    )(co, fo, so, gc, Wo.reshape(H, DH, d).astype(BF16))

    return out.reshape(b, n, d)


# speedup vs baseline: 1.3264x; 1.3264x over previous
import jax
import jax.numpy as jnp
from jax.experimental import pallas as pl
from jax.experimental.pallas import tpu as pltpu

B, N, D = 1, 2048, 768
H, KVH, DH = 12, 12, 64
BLK, STRIDE = 16, 8
SELBLK, NSEL = 16, 4
WIN = 64
SCALE = DH ** -0.5
NCB = (N - BLK) // STRIDE + 1
NSB = N // SELBLK
TQ = 256
BF16 = jnp.bfloat16
F32 = jnp.float32


def _bdot(a, b, dims=None):
    if dims is None:
        dims = (((a.ndim - 1,), (0,)), ((), ()))
    return jax.lax.dot_general(a.astype(BF16), b.astype(BF16), dims,
                               preferred_element_type=F32)


def _proj_kernel(x_ref, g_ref, wq_ref, wk_ref, wv_ref, wc_ref, bc_ref,
                 qb_ref, kb_ref, vb_ref, k_ref, v_ref, gc_ref):
    x = x_ref[:]
    xn = x * jax.lax.rsqrt(jnp.mean(x * x, axis=-1, keepdims=True) + 1e-6)
    xn = xn * g_ref[:]
    xnb = xn.astype(BF16)
    dd = lambda w: jax.lax.dot_general(xnb, w, (((1,), (0,)), ((), ())),
                                       preferred_element_type=F32)
    q = dd(wq_ref[:])
    k = dd(wk_ref[:])
    v = dd(wv_ref[:])
    qb_ref[:] = q.astype(BF16)
    kb_ref[:] = k.astype(BF16)
    vb_ref[:] = v.astype(BF16)
    k_ref[:] = k
    v_ref[:] = v
    gc_ref[:] = jax.nn.sigmoid(dd(wc_ref[:]) + bc_ref[:])


def _compress_kernel(kr_ref, vr_ref, kpe_ref, vpe_ref, wck_ref, wcv_ref,
                     mem_ref, ckf_ref, cvf_ref):
    k8 = kr_ref[0]
    v8 = vr_ref[0]
    half = STRIDE * DH
    kpe = kpe_ref[0]
    vpe = vpe_ref[0]
    k8s = jnp.concatenate([k8[1:], k8[:1]], axis=0)
    v8s = jnp.concatenate([v8[1:], v8[:1]], axis=0)
    kbf = jnp.concatenate([k8 + kpe[:, :half], k8s + kpe[:, half:]], axis=1)
    vbf = jnp.concatenate([v8 + vpe[:, :half], v8s + vpe[:, half:]], axis=1)
    ck = _bdot(kbf, wck_ref[0])
    cv = _bdot(vbf, wcv_ref[0])
    ckf_ref[0] = jnp.concatenate([mem_ref[0, 0], ck[:NCB]], axis=0)
    cvf_ref[0] = jnp.concatenate([mem_ref[1, 0], cv[:NCB]], axis=0)


def _attn_kernel(q_ref, k_ref, v_ref, ckf_ref, cvf_ref,
                 co_ref, fo_ref, so_ref, *, nk, qoff):
    i = qoff + pl.program_id(1)
    W = nk * TQ
    qb = q_ref[0]
    ckf = ckf_ref[0]
    cvf = cvf_ref[0]
    t = i * TQ + jax.lax.broadcasted_iota(jnp.int32, (TQ, 1), 0)

    cs = jax.lax.dot_general(qb, ckf.astype(BF16), (((1,), (1,)), ((), ())),
                             preferred_element_type=F32) * SCALE
    jc = jax.lax.broadcasted_iota(jnp.int32, (TQ, NCB + 1), 1)
    cmask = (jc == 0) | ((jc - 1) * STRIDE + BLK - 1 <= t)
    cs = jnp.where(cmask, cs, -1e30)
    cm = jnp.max(cs, axis=-1, keepdims=True)
    ce = jnp.exp(cs - cm)
    cp = ce / jnp.sum(ce, axis=-1, keepdims=True)
    co_ref[0] = _bdot(cp, cvf)

    cpr = jnp.concatenate([cp[:, 1:], cp[:, :1]], axis=1)
    impw = cp + cpr
    lane = jax.lax.broadcasted_iota(jnp.int32, (TQ, NCB + 1), 1)
    odd = (lane % 2) == 1
    impw = jnp.where(odd, impw, -1.0)
    impw = jnp.where(lane == NCB, cp[:, NCB:NCB + 1], impw)
    sels = []
    for _ in range(NSEL):
        idx = jnp.argmax(impw, axis=-1, keepdims=True).astype(jnp.int32)
        sels.append(jax.lax.shift_right_logical(idx, 1))
        impw = jnp.where(lane == idx, -2.0, impw)

    tb = t // SELBLK
    kk = k_ref[0]
    vv = v_ref[0]
    s = jax.lax.dot_general(qb, kk, (((1,), (1,)), ((), ())),
                            preferred_element_type=F32) * SCALE
    jcol = jax.lax.broadcasted_iota(jnp.int32, (TQ, W), 1)
    jb = jcol // SELBLK
    fmask = (jb == sels[0]) | (jb == sels[1]) | (jb == sels[2]) \
        | (jb == sels[3]) | (jb == tb)
    fmask = fmask & (jcol <= t)
    fs = jnp.where(fmask, s, -1e30)
    fm = jnp.max(fs, axis=-1, keepdims=True)
    fe = jnp.exp(fs - fm)
    fp = (fe / jnp.sum(fe, axis=-1, keepdims=True)).astype(BF16)
    fo_ref[0] = jax.lax.dot_general(fp, vv, (((1,), (0,)), ((), ())),
                                    preferred_element_type=F32)

    start = jnp.maximum(i - 1, 0) * TQ
    kw = k_ref[0, pl.ds(start, 2 * TQ), :]
    vw = v_ref[0, pl.ds(start, 2 * TQ), :]
    sw = jax.lax.dot_general(qb, kw, (((1,), (1,)), ((), ())),
                             preferred_element_type=F32) * SCALE
    jcw = start + jax.lax.broadcasted_iota(jnp.int32, (TQ, 2 * TQ), 1)
    smask = (jcw <= t) & (t - jcw < WIN)
    ws = jnp.where(smask, sw, -1e30)
    wm = jnp.max(ws, axis=-1, keepdims=True)
    we = jnp.where(smask, jnp.exp(ws - wm), 0.0)
    wp = we / jnp.sum(we, axis=-1, keepdims=True)
    so_ref[0] = _bdot(wp, vw)


def _out_kernel(co_ref, fo_ref, so_ref, gc_ref, wo_ref, o_ref):
    gc = gc_ref[:]
    acc = jnp.zeros((TQ, D), F32)
    for h in range(H):
        comb = gc[:, h:h + 1] * co_ref[h] \
            + gc[:, H + h:H + h + 1] * fo_ref[h] \
            + gc[:, 2 * H + h:2 * H + h + 1] * so_ref[h]
        acc = acc + _bdot(comb, wo_ref[h])
    o_ref[:] = acc


def kernel(x, g_norm, Wq, Wk, Wv, k_pe, v_pe, Wck, Wcv, mem_kv,
           W_comb, b_comb, Wo):
    b, n, d = x.shape
    x2 = x.reshape(n, d)
    g2 = g_norm.reshape(1, d)
    b2 = b_comb.reshape(1, 3 * H)

    qb, kb, vb, k, v, gc = pl.pallas_call(
        _proj_kernel,
        grid=(n // TQ,),
        in_specs=[
            pl.BlockSpec((TQ, d), lambda i: (i, 0)),
            pl.BlockSpec((1, d), lambda i: (0, 0)),
            pl.BlockSpec((d, H * DH), lambda i: (0, 0)),
            pl.BlockSpec((d, KVH * DH), lambda i: (0, 0)),
            pl.BlockSpec((d, KVH * DH), lambda i: (0, 0)),
            pl.BlockSpec((d, 3 * H), lambda i: (0, 0)),
            pl.BlockSpec((1, 3 * H), lambda i: (0, 0)),
        ],
        out_specs=[
            pl.BlockSpec((TQ, H * DH), lambda i: (i, 0)),
            pl.BlockSpec((TQ, KVH * DH), lambda i: (i, 0)),
            pl.BlockSpec((TQ, KVH * DH), lambda i: (i, 0)),
            pl.BlockSpec((TQ, KVH * DH), lambda i: (i, 0)),
            pl.BlockSpec((TQ, KVH * DH), lambda i: (i, 0)),
            pl.BlockSpec((TQ, 3 * H), lambda i: (i, 0)),
        ],
        out_shape=[
            jax.ShapeDtypeStruct((n, H * DH), BF16),
            jax.ShapeDtypeStruct((n, KVH * DH), BF16),
            jax.ShapeDtypeStruct((n, KVH * DH), BF16),
            jax.ShapeDtypeStruct((n, KVH * DH), F32),
            jax.ShapeDtypeStruct((n, KVH * DH), F32),
            jax.ShapeDtypeStruct((n, 3 * H), F32),
        ],
    )(x2, g2, Wq.astype(BF16), Wk.astype(BF16), Wv.astype(BF16),
      W_comb.astype(BF16), b2)

    qh = qb.reshape(n, H, DH).transpose(1, 0, 2)
    kh = kb.reshape(n, KVH, DH).transpose(1, 0, 2)
    vh = vb.reshape(n, KVH, DH).transpose(1, 0, 2)
    kr = k.reshape(n, KVH, DH).transpose(1, 0, 2) \
        .reshape(KVH, n // STRIDE, STRIDE * DH)
    vr = v.reshape(n, KVH, DH).transpose(1, 0, 2) \
        .reshape(KVH, n // STRIDE, STRIDE * DH)
    kpe2 = k_pe.reshape(KVH, 1, BLK * DH)
    vpe2 = v_pe.reshape(KVH, 1, BLK * DH)
    mem2 = mem_kv.reshape(2, KVH, 1, DH)

    ckf, cvf = pl.pallas_call(
        _compress_kernel,
        grid=(KVH,),
        in_specs=[
            pl.BlockSpec((1, n // STRIDE, STRIDE * DH), lambda h: (h, 0, 0)),
            pl.BlockSpec((1, n // STRIDE, STRIDE * DH), lambda h: (h, 0, 0)),
            pl.BlockSpec((1, 1, BLK * DH), lambda h: (h, 0, 0)),
            pl.BlockSpec((1, 1, BLK * DH), lambda h: (h, 0, 0)),
            pl.BlockSpec((1, BLK * DH, DH), lambda h: (h, 0, 0)),
            pl.BlockSpec((1, BLK * DH, DH), lambda h: (h, 0, 0)),
            pl.BlockSpec((2, 1, 1, DH), lambda h: (0, h, 0, 0)),
        ],
        out_specs=[
            pl.BlockSpec((1, NCB + 1, DH), lambda h: (h, 0, 0)),
            pl.BlockSpec((1, NCB + 1, DH), lambda h: (h, 0, 0)),
        ],
        out_shape=[
            jax.ShapeDtypeStruct((KVH, NCB + 1, DH), F32),
            jax.ShapeDtypeStruct((KVH, NCB + 1, DH), F32),
        ],
    )(kr, vr, kpe2, vpe2, Wck.astype(BF16), Wcv.astype(BF16), mem2)

    import functools
    parts = []
    for vi in range(4):
        qoff = 2 * vi
        nk = 2 * vi + 2
        W = nk * TQ
        part = pl.pallas_call(
            functools.partial(_attn_kernel, nk=nk, qoff=qoff),
            grid=(H, 2),
            in_specs=[
                pl.BlockSpec((1, TQ, DH), lambda h, i, q0=qoff: (h, q0 + i, 0)),
                pl.BlockSpec((1, W, DH), lambda h, i: (h, 0, 0)),
                pl.BlockSpec((1, W, DH), lambda h, i: (h, 0, 0)),
                pl.BlockSpec((1, NCB + 1, DH), lambda h, i: (h, 0, 0)),
                pl.BlockSpec((1, NCB + 1, DH), lambda h, i: (h, 0, 0)),
            ],
            out_specs=[
                pl.BlockSpec((1, TQ, DH), lambda h, i: (h, i, 0)),
                pl.BlockSpec((1, TQ, DH), lambda h, i: (h, i, 0)),
                pl.BlockSpec((1, TQ, DH), lambda h, i: (h, i, 0)),
            ],
            out_shape=[
                jax.ShapeDtypeStruct((H, 2 * TQ, DH), F32),
                jax.ShapeDtypeStruct((H, 2 * TQ, DH), F32),
                jax.ShapeDtypeStruct((H, 2 * TQ, DH), F32),
            ],
        )(qh, kh, vh, ckf, cvf)
        parts.append(part)
    co = jnp.concatenate([p[0] for p in parts], axis=1)
    fo = jnp.concatenate([p[1] for p in parts], axis=1)
    so = jnp.concatenate([p[2] for p in parts], axis=1)

    out = pl.pallas_call(
        _out_kernel,
        grid=(n // TQ,),
        in_specs=[
            pl.BlockSpec((H, TQ, DH), lambda i: (0, i, 0)),
            pl.BlockSpec((H, TQ, DH), lambda i: (0, i, 0)),
            pl.BlockSpec((H, TQ, DH), lambda i: (0, i, 0)),
            pl.BlockSpec((TQ, 3 * H), lambda i: (i, 0)),
            pl.BlockSpec((H, DH, d), lambda i: (0, 0, 0)),
        ],
        out_specs=pl.BlockSpec((TQ, d), lambda i: (i, 0)),
        out_shape=jax.ShapeDtypeStruct((n, d), F32),
    )(co, fo, so, gc, Wo.reshape(H, DH, d).astype(BF16))

    return out.reshape(b, n, d)


# speedup vs baseline: 1.4483x; 1.0919x over previous
import jax
import jax.numpy as jnp
from jax.experimental import pallas as pl
from jax.experimental.pallas import tpu as pltpu

B, N, D = 1, 2048, 768
H, KVH, DH = 12, 12, 64
BLK, STRIDE = 16, 8
SELBLK, NSEL = 16, 4
WIN = 64
SCALE = DH ** -0.5
NCB = (N - BLK) // STRIDE + 1
NSB = N // SELBLK
TQ = 256
BF16 = jnp.bfloat16
F32 = jnp.float32


def _bdot(a, b, dims=None):
    if dims is None:
        dims = (((a.ndim - 1,), (0,)), ((), ()))
    return jax.lax.dot_general(a.astype(BF16), b.astype(BF16), dims,
                               preferred_element_type=F32)


def _proj_kernel(x_ref, g_ref, wq_ref, wk_ref, wv_ref, wc_ref, bc_ref,
                 qb_ref, kb_ref, vb_ref, k_ref, v_ref, gc_ref):
    x = x_ref[:]
    xn = x * jax.lax.rsqrt(jnp.mean(x * x, axis=-1, keepdims=True) + 1e-6)
    xn = xn * g_ref[:]
    xnb = xn.astype(BF16)
    dd = lambda w: jax.lax.dot_general(xnb, w, (((1,), (0,)), ((), ())),
                                       preferred_element_type=F32)
    q = dd(wq_ref[:])
    k = dd(wk_ref[:])
    v = dd(wv_ref[:])
    per_head = lambda a: jnp.swapaxes(a.reshape(TQ, H, DH), 0, 1)
    qb_ref[:] = per_head(q.astype(BF16))
    kb_ref[:] = per_head(k.astype(BF16))
    vb_ref[:] = per_head(v.astype(BF16))
    k_ref[:] = per_head(k)
    v_ref[:] = per_head(v)
    gc_ref[:] = jax.nn.sigmoid(dd(wc_ref[:]) + bc_ref[:])


def _compress_kernel(kr_ref, vr_ref, kpe_ref, vpe_ref, wck_ref, wcv_ref,
                     mem_ref, ckf_ref, cvf_ref):
    k8 = kr_ref[0]
    v8 = vr_ref[0]
    half = STRIDE * DH
    kpe = kpe_ref[0]
    vpe = vpe_ref[0]
    k8s = jnp.concatenate([k8[1:], k8[:1]], axis=0)
    v8s = jnp.concatenate([v8[1:], v8[:1]], axis=0)
    kbf = jnp.concatenate([k8 + kpe[:, :half], k8s + kpe[:, half:]], axis=1)
    vbf = jnp.concatenate([v8 + vpe[:, :half], v8s + vpe[:, half:]], axis=1)
    ck = _bdot(kbf, wck_ref[0])
    cv = _bdot(vbf, wcv_ref[0])
    ckf_ref[0] = jnp.concatenate([mem_ref[0, 0], ck[:NCB]], axis=0)
    cvf_ref[0] = jnp.concatenate([mem_ref[1, 0], cv[:NCB]], axis=0)


def _attn_kernel(q_ref, k_ref, v_ref, ckf_ref, cvf_ref,
                 co_ref, fo_ref, so_ref, *, nk, qoff):
    i = qoff + pl.program_id(1)
    W = nk * TQ
    qb = q_ref[0]
    ckf = ckf_ref[0]
    cvf = cvf_ref[0]
    t = i * TQ + jax.lax.broadcasted_iota(jnp.int32, (TQ, 1), 0)

    cs = jax.lax.dot_general(qb, ckf.astype(BF16), (((1,), (1,)), ((), ())),
                             preferred_element_type=F32) * SCALE
    jc = jax.lax.broadcasted_iota(jnp.int32, (TQ, NCB + 1), 1)
    cmask = (jc == 0) | ((jc - 1) * STRIDE + BLK - 1 <= t)
    cs = jnp.where(cmask, cs, -1e30)
    cm = jnp.max(cs, axis=-1, keepdims=True)
    ce = jnp.exp(cs - cm)
    cp = ce / jnp.sum(ce, axis=-1, keepdims=True)
    co_ref[0] = _bdot(cp, cvf)

    cpr = jnp.concatenate([cp[:, 1:], cp[:, :1]], axis=1)
    impw = cp + cpr
    lane = jax.lax.broadcasted_iota(jnp.int32, (TQ, NCB + 1), 1)
    odd = (lane % 2) == 1
    impw = jnp.where(odd, impw, -1.0)
    impw = jnp.where(lane == NCB, cp[:, NCB:NCB + 1], impw)
    sels = []
    for _ in range(NSEL):
        idx = jnp.argmax(impw, axis=-1, keepdims=True).astype(jnp.int32)
        sels.append(jax.lax.shift_right_logical(idx, 1))
        impw = jnp.where(lane == idx, -2.0, impw)

    tb = t // SELBLK
    kk = k_ref[0]
    vv = v_ref[0]
    s = jax.lax.dot_general(qb, kk, (((1,), (1,)), ((), ())),
                            preferred_element_type=F32) * SCALE
    jcol = jax.lax.broadcasted_iota(jnp.int32, (TQ, W), 1)
    jb = jcol // SELBLK
    fmask = (jb == sels[0]) | (jb == sels[1]) | (jb == sels[2]) \
        | (jb == sels[3]) | (jb == tb)
    fmask = fmask & (jcol <= t)
    fs = jnp.where(fmask, s, -1e30)
    fm = jnp.max(fs, axis=-1, keepdims=True)
    fe = jnp.exp(fs - fm)
    fp = (fe / jnp.sum(fe, axis=-1, keepdims=True)).astype(BF16)
    fo_ref[0] = jax.lax.dot_general(fp, vv, (((1,), (0,)), ((), ())),
                                    preferred_element_type=F32)

    start = jnp.maximum(i - 1, 0) * TQ
    kw = k_ref[0, pl.ds(start, 2 * TQ), :]
    vw = v_ref[0, pl.ds(start, 2 * TQ), :]
    sw = jax.lax.dot_general(qb, kw, (((1,), (1,)), ((), ())),
                             preferred_element_type=F32) * SCALE
    jcw = start + jax.lax.broadcasted_iota(jnp.int32, (TQ, 2 * TQ), 1)
    smask = (jcw <= t) & (t - jcw < WIN)
    ws = jnp.where(smask, sw, -1e30)
    wm = jnp.max(ws, axis=-1, keepdims=True)
    we = jnp.where(smask, jnp.exp(ws - wm), 0.0)
    wp = we / jnp.sum(we, axis=-1, keepdims=True)
    so_ref[0] = _bdot(wp, vw)


def _out_kernel(co_ref, fo_ref, so_ref, gc_ref, wo_ref, o_ref):
    gc = gc_ref[:]
    acc = jnp.zeros((TQ, D), F32)
    for h in range(H):
        comb = gc[:, h:h + 1] * co_ref[h] \
            + gc[:, H + h:H + h + 1] * fo_ref[h] \
            + gc[:, 2 * H + h:2 * H + h + 1] * so_ref[h]
        acc = acc + _bdot(comb, wo_ref[h])
    o_ref[:] = acc


def kernel(x, g_norm, Wq, Wk, Wv, k_pe, v_pe, Wck, Wcv, mem_kv,
           W_comb, b_comb, Wo):
    b, n, d = x.shape
    x2 = x.reshape(n, d)
    g2 = g_norm.reshape(1, d)
    b2 = b_comb.reshape(1, 3 * H)

    qb, kb, vb, k, v, gc = pl.pallas_call(
        _proj_kernel,
        grid=(n // TQ,),
        in_specs=[
            pl.BlockSpec((TQ, d), lambda i: (i, 0)),
            pl.BlockSpec((1, d), lambda i: (0, 0)),
            pl.BlockSpec((d, H * DH), lambda i: (0, 0)),
            pl.BlockSpec((d, KVH * DH), lambda i: (0, 0)),
            pl.BlockSpec((d, KVH * DH), lambda i: (0, 0)),
            pl.BlockSpec((d, 3 * H), lambda i: (0, 0)),
            pl.BlockSpec((1, 3 * H), lambda i: (0, 0)),
        ],
        out_specs=[
            pl.BlockSpec((H, TQ, DH), lambda i: (0, i, 0)),
            pl.BlockSpec((KVH, TQ, DH), lambda i: (0, i, 0)),
            pl.BlockSpec((KVH, TQ, DH), lambda i: (0, i, 0)),
            pl.BlockSpec((KVH, TQ, DH), lambda i: (0, i, 0)),
            pl.BlockSpec((KVH, TQ, DH), lambda i: (0, i, 0)),
            pl.BlockSpec((TQ, 3 * H), lambda i: (i, 0)),
        ],
        out_shape=[
            jax.ShapeDtypeStruct((H, n, DH), BF16),
            jax.ShapeDtypeStruct((KVH, n, DH), BF16),
            jax.ShapeDtypeStruct((KVH, n, DH), BF16),
            jax.ShapeDtypeStruct((KVH, n, DH), F32),
            jax.ShapeDtypeStruct((KVH, n, DH), F32),
            jax.ShapeDtypeStruct((n, 3 * H), F32),
        ],
    )(x2, g2, Wq.astype(BF16), Wk.astype(BF16), Wv.astype(BF16),
      W_comb.astype(BF16), b2)

    qh, kh, vh = qb, kb, vb
    kr = k.reshape(KVH, n // STRIDE, STRIDE * DH)
    vr = v.reshape(KVH, n // STRIDE, STRIDE * DH)
    kpe2 = k_pe.reshape(KVH, 1, BLK * DH)
    vpe2 = v_pe.reshape(KVH, 1, BLK * DH)
    mem2 = mem_kv.reshape(2, KVH, 1, DH)

    ckf, cvf = pl.pallas_call(
        _compress_kernel,
        grid=(KVH,),
        in_specs=[
            pl.BlockSpec((1, n // STRIDE, STRIDE * DH), lambda h: (h, 0, 0)),
            pl.BlockSpec((1, n // STRIDE, STRIDE * DH), lambda h: (h, 0, 0)),
            pl.BlockSpec((1, 1, BLK * DH), lambda h: (h, 0, 0)),
            pl.BlockSpec((1, 1, BLK * DH), lambda h: (h, 0, 0)),
            pl.BlockSpec((1, BLK * DH, DH), lambda h: (h, 0, 0)),
            pl.BlockSpec((1, BLK * DH, DH), lambda h: (h, 0, 0)),
            pl.BlockSpec((2, 1, 1, DH), lambda h: (0, h, 0, 0)),
        ],
        out_specs=[
            pl.BlockSpec((1, NCB + 1, DH), lambda h: (h, 0, 0)),
            pl.BlockSpec((1, NCB + 1, DH), lambda h: (h, 0, 0)),
        ],
        out_shape=[
            jax.ShapeDtypeStruct((KVH, NCB + 1, DH), F32),
            jax.ShapeDtypeStruct((KVH, NCB + 1, DH), F32),
        ],
    )(kr, vr, kpe2, vpe2, Wck.astype(BF16), Wcv.astype(BF16), mem2)

    import functools
    parts = []
    for vi in range(4):
        qoff = 2 * vi
        nk = 2 * vi + 2
        W = nk * TQ
        part = pl.pallas_call(
            functools.partial(_attn_kernel, nk=nk, qoff=qoff),
            grid=(H, 2),
            in_specs=[
                pl.BlockSpec((1, TQ, DH), lambda h, i, q0=qoff: (h, q0 + i, 0)),
                pl.BlockSpec((1, W, DH), lambda h, i: (h, 0, 0)),
                pl.BlockSpec((1, W, DH), lambda h, i: (h, 0, 0)),
                pl.BlockSpec((1, NCB + 1, DH), lambda h, i: (h, 0, 0)),
                pl.BlockSpec((1, NCB + 1, DH), lambda h, i: (h, 0, 0)),
            ],
            out_specs=[
                pl.BlockSpec((1, TQ, DH), lambda h, i: (h, i, 0)),
                pl.BlockSpec((1, TQ, DH), lambda h, i: (h, i, 0)),
                pl.BlockSpec((1, TQ, DH), lambda h, i: (h, i, 0)),
            ],
            out_shape=[
                jax.ShapeDtypeStruct((H, 2 * TQ, DH), F32),
                jax.ShapeDtypeStruct((H, 2 * TQ, DH), F32),
                jax.ShapeDtypeStruct((H, 2 * TQ, DH), F32),
            ],
        )(qh, kh, vh, ckf, cvf)
        parts.append(part)
    co = jnp.concatenate([p[0] for p in parts], axis=1)
    fo = jnp.concatenate([p[1] for p in parts], axis=1)
    so = jnp.concatenate([p[2] for p in parts], axis=1)

    out = pl.pallas_call(
        _out_kernel,
        grid=(n // TQ,),
        in_specs=[
            pl.BlockSpec((H, TQ, DH), lambda i: (0, i, 0)),
            pl.BlockSpec((H, TQ, DH), lambda i: (0, i, 0)),
            pl.BlockSpec((H, TQ, DH), lambda i: (0, i, 0)),
            pl.BlockSpec((TQ, 3 * H), lambda i: (i, 0)),
            pl.BlockSpec((H, DH, d), lambda i: (0, 0, 0)),
        ],
        out_specs=pl.BlockSpec((TQ, d), lambda i: (i, 0)),
        out_shape=jax.ShapeDtypeStruct((n, d), F32),
    )(co, fo, so, gc, Wo.reshape(H, DH, d).astype(BF16))

    return out.reshape(b, n, d)


# speedup vs baseline: 1.6902x; 1.1670x over previous
import jax
import jax.numpy as jnp
from jax.experimental import pallas as pl
from jax.experimental.pallas import tpu as pltpu

B, N, D = 1, 2048, 768
H, KVH, DH = 12, 12, 64
BLK, STRIDE = 16, 8
SELBLK, NSEL = 16, 4
WIN = 64
SCALE = DH ** -0.5
NCB = (N - BLK) // STRIDE + 1
NSB = N // SELBLK
TQ = 256
BF16 = jnp.bfloat16
F32 = jnp.float32


def _bdot(a, b, dims=None):
    if dims is None:
        dims = (((a.ndim - 1,), (0,)), ((), ()))
    return jax.lax.dot_general(a.astype(BF16), b.astype(BF16), dims,
                               preferred_element_type=F32)


def _proj_kernel(x_ref, g_ref, wq_ref, wk_ref, wv_ref, wc_ref, bc_ref,
                 qb_ref, kb_ref, vb_ref, k_ref, v_ref, gc_ref):
    x = x_ref[:]
    xn = x * jax.lax.rsqrt(jnp.mean(x * x, axis=-1, keepdims=True) + 1e-6)
    xn = xn * g_ref[:]
    xnb = xn.astype(BF16)
    dd = lambda w: jax.lax.dot_general(xnb, w, (((1,), (0,)), ((), ())),
                                       preferred_element_type=F32)
    q = dd(wq_ref[:])
    k = dd(wk_ref[:])
    v = dd(wv_ref[:])
    per_head = lambda a: jnp.swapaxes(a.reshape(TQ, H, DH), 0, 1)
    qb_ref[:] = per_head((q * SCALE).astype(BF16))
    kb_ref[:] = per_head(k.astype(BF16))
    vb_ref[:] = per_head(v.astype(BF16))
    k_ref[:] = per_head(k)
    v_ref[:] = per_head(v)
    gc_ref[:] = jax.nn.sigmoid(dd(wc_ref[:]) + bc_ref[:])


def _compress_kernel(kr_ref, vr_ref, kpe_ref, vpe_ref, wck_ref, wcv_ref,
                     mem_ref, ckf_ref, cvf_ref):
    k8 = kr_ref[0]
    v8 = vr_ref[0]
    half = STRIDE * DH
    kpe = kpe_ref[0]
    vpe = vpe_ref[0]
    k8s = jnp.concatenate([k8[1:], k8[:1]], axis=0)
    v8s = jnp.concatenate([v8[1:], v8[:1]], axis=0)
    kbf = jnp.concatenate([k8 + kpe[:, :half], k8s + kpe[:, half:]], axis=1)
    vbf = jnp.concatenate([v8 + vpe[:, :half], v8s + vpe[:, half:]], axis=1)
    ck = _bdot(kbf, wck_ref[0])
    cv = _bdot(vbf, wcv_ref[0])
    ckf_ref[0] = jnp.concatenate([mem_ref[0, 0], ck[:NCB]], axis=0)
    cvf_ref[0] = jnp.concatenate([mem_ref[1, 0], cv[:NCB]], axis=0)


def _attn_kernel(q_ref, k_ref, v_ref, ckf_ref, cvf_ref, e_ref,
                 co_ref, fo_ref, so_ref, *, vi):
    W = (vi + 1) * TQ
    BODY = W - TQ
    qb = q_ref[0]
    ckf = ckf_ref[0]
    cvf = cvf_ref[0]
    t = vi * TQ + jax.lax.broadcasted_iota(jnp.int32, (TQ, 1), 0)

    cs = jax.lax.dot_general(qb, ckf.astype(BF16), (((1,), (1,)), ((), ())),
                             preferred_element_type=F32)
    jc = jax.lax.broadcasted_iota(jnp.int32, (TQ, NCB + 1), 1)
    cmask = (jc == 0) | ((jc - 1) * STRIDE + BLK - 1 <= t)
    cs = jnp.where(cmask, cs, -1e30)
    cm = jnp.max(cs, axis=-1, keepdims=True)
    ce = jnp.exp(cs - cm)
    cp = ce / jnp.sum(ce, axis=-1, keepdims=True)
    co_ref[0] = _bdot(cp, cvf)

    cpr = jnp.concatenate([cp[:, 1:], cp[:, :1]], axis=1)
    impw = cp + cpr
    lane = jax.lax.broadcasted_iota(jnp.int32, (TQ, NCB + 1), 1)
    odd = (lane % 2) == 1
    impw = jnp.where(odd, impw, -1.0)
    impw = jnp.where(lane == NCB, cp[:, NCB:NCB + 1], impw)
    sels = []
    for _ in range(NSEL):
        idx = jnp.argmax(impw, axis=-1, keepdims=True).astype(jnp.int32)
        sels.append(jax.lax.shift_right_logical(idx, 1))
        impw = jnp.where(lane == idx, -2.0, impw)

    tb = t // SELBLK
    kk = k_ref[0]
    vv = v_ref[0]
    s = jax.lax.dot_general(qb, kk, (((1,), (1,)), ((), ())),
                            preferred_element_type=F32)
    rr = jax.lax.broadcasted_iota(jnp.int32, (TQ, TQ), 0)
    cc = jax.lax.broadcasted_iota(jnp.int32, (TQ, TQ), 1)
    jbq = (BODY + cc) // SELBLK
    own01 = (cc <= rr) & (jbq == tb)
    selc = ((jbq == sels[0]) | (jbq == sels[1]) | (jbq == sels[2])
            | (jbq == sels[3])) & (jbq < tb)
    chunkm = jnp.where(own01 | selc, s[:, BODY:], -1e30)
    if vi > 0:
        sblk = jax.lax.broadcasted_iota(jnp.int32, (TQ, BODY // SELBLK), 1)
        mblk = ((sblk == sels[0]) | (sblk == sels[1]) | (sblk == sels[2])
                | (sblk == sels[3]))
        m01 = jax.lax.dot_general(mblk.astype(BF16), e_ref[:],
                                  (((1,), (0,)), ((), ())),
                                  preferred_element_type=F32)
        fs = jnp.concatenate([s[:, :BODY] + (m01 * 1e30 - 1e30), chunkm],
                             axis=1)
    else:
        fs = chunkm
    fm = jnp.max(fs, axis=-1, keepdims=True)
    fe = jnp.exp(fs - fm)
    fp = (fe / jnp.sum(fe, axis=-1, keepdims=True)).astype(BF16)
    fo_ref[0] = jax.lax.dot_general(fp, vv, (((1,), (0,)), ((), ())),
                                    preferred_element_type=F32)

    ws0 = max(vi - 1, 0) * TQ
    WSL = W - ws0
    kw = kk[ws0:, :]
    sw = jax.lax.dot_general(qb, kw, (((1,), (1,)), ((), ())),
                             preferred_element_type=F32)
    jcw = ws0 + jax.lax.broadcasted_iota(jnp.int32, (TQ, WSL), 1)
    smask = (jcw <= t) & (t - jcw < WIN)
    wsv = jnp.where(smask, sw, -1e30)
    wm = jnp.max(wsv, axis=-1, keepdims=True)
    we = jnp.where(smask, jnp.exp(wsv - wm), 0.0)
    wp = we / jnp.sum(we, axis=-1, keepdims=True)
    so_ref[0] = _bdot(wp, vv[ws0:, :])


def _out_kernel(co_ref, fo_ref, so_ref, gc_ref, wo_ref, o_ref):
    gc = gc_ref[:]
    acc = jnp.zeros((TQ, D), F32)
    for h in range(H):
        comb = gc[:, h:h + 1] * co_ref[h] \
            + gc[:, H + h:H + h + 1] * fo_ref[h] \
            + gc[:, 2 * H + h:2 * H + h + 1] * so_ref[h]
        acc = acc + _bdot(comb, wo_ref[h])
    o_ref[:] = acc


def kernel(x, g_norm, Wq, Wk, Wv, k_pe, v_pe, Wck, Wcv, mem_kv,
           W_comb, b_comb, Wo):
    b, n, d = x.shape
    x2 = x.reshape(n, d)
    g2 = g_norm.reshape(1, d)
    b2 = b_comb.reshape(1, 3 * H)

    qb, kb, vb, k, v, gc = pl.pallas_call(
        _proj_kernel,
        grid=(n // TQ,),
        in_specs=[
            pl.BlockSpec((TQ, d), lambda i: (i, 0)),
            pl.BlockSpec((1, d), lambda i: (0, 0)),
            pl.BlockSpec((d, H * DH), lambda i: (0, 0)),
            pl.BlockSpec((d, KVH * DH), lambda i: (0, 0)),
            pl.BlockSpec((d, KVH * DH), lambda i: (0, 0)),
            pl.BlockSpec((d, 3 * H), lambda i: (0, 0)),
            pl.BlockSpec((1, 3 * H), lambda i: (0, 0)),
        ],
        out_specs=[
            pl.BlockSpec((H, TQ, DH), lambda i: (0, i, 0)),
            pl.BlockSpec((KVH, TQ, DH), lambda i: (0, i, 0)),
            pl.BlockSpec((KVH, TQ, DH), lambda i: (0, i, 0)),
            pl.BlockSpec((KVH, TQ, DH), lambda i: (0, i, 0)),
            pl.BlockSpec((KVH, TQ, DH), lambda i: (0, i, 0)),
            pl.BlockSpec((TQ, 3 * H), lambda i: (i, 0)),
        ],
        out_shape=[
            jax.ShapeDtypeStruct((H, n, DH), BF16),
            jax.ShapeDtypeStruct((KVH, n, DH), BF16),
            jax.ShapeDtypeStruct((KVH, n, DH), BF16),
            jax.ShapeDtypeStruct((KVH, n, DH), F32),
            jax.ShapeDtypeStruct((KVH, n, DH), F32),
            jax.ShapeDtypeStruct((n, 3 * H), F32),
        ],
    )(x2, g2, Wq.astype(BF16), Wk.astype(BF16), Wv.astype(BF16),
      W_comb.astype(BF16), b2)

    qh, kh, vh = qb, kb, vb
    kr = k.reshape(KVH, n // STRIDE, STRIDE * DH)
    vr = v.reshape(KVH, n // STRIDE, STRIDE * DH)
    kpe2 = k_pe.reshape(KVH, 1, BLK * DH)
    vpe2 = v_pe.reshape(KVH, 1, BLK * DH)
    mem2 = mem_kv.reshape(2, KVH, 1, DH)

    ckf, cvf = pl.pallas_call(
        _compress_kernel,
        grid=(KVH,),
        in_specs=[
            pl.BlockSpec((1, n // STRIDE, STRIDE * DH), lambda h: (h, 0, 0)),
            pl.BlockSpec((1, n // STRIDE, STRIDE * DH), lambda h: (h, 0, 0)),
            pl.BlockSpec((1, 1, BLK * DH), lambda h: (h, 0, 0)),
            pl.BlockSpec((1, 1, BLK * DH), lambda h: (h, 0, 0)),
            pl.BlockSpec((1, BLK * DH, DH), lambda h: (h, 0, 0)),
            pl.BlockSpec((1, BLK * DH, DH), lambda h: (h, 0, 0)),
            pl.BlockSpec((2, 1, 1, DH), lambda h: (0, h, 0, 0)),
        ],
        out_specs=[
            pl.BlockSpec((1, NCB + 1, DH), lambda h: (h, 0, 0)),
            pl.BlockSpec((1, NCB + 1, DH), lambda h: (h, 0, 0)),
        ],
        out_shape=[
            jax.ShapeDtypeStruct((KVH, NCB + 1, DH), F32),
            jax.ShapeDtypeStruct((KVH, NCB + 1, DH), F32),
        ],
    )(kr, vr, kpe2, vpe2, Wck.astype(BF16), Wcv.astype(BF16), mem2)

    import functools
    expander = (jnp.arange(n)[None, :] // SELBLK
                == jnp.arange(NSB)[:, None]).astype(BF16)
    parts = []
    for vi in range(n // TQ):
        W = (vi + 1) * TQ
        BODY = W - TQ
        in_specs = [
            pl.BlockSpec((1, TQ, DH), lambda h, v=vi: (h, v, 0)),
            pl.BlockSpec((1, W, DH), lambda h: (h, 0, 0)),
            pl.BlockSpec((1, W, DH), lambda h: (h, 0, 0)),
            pl.BlockSpec((1, NCB + 1, DH), lambda h: (h, 0, 0)),
            pl.BlockSpec((1, NCB + 1, DH), lambda h: (h, 0, 0)),
        ]
        args = [qh, kh, vh, ckf, cvf]
        if vi > 0:
            in_specs.append(
                pl.BlockSpec((BODY // SELBLK, BODY), lambda h: (0, 0)))
            args.append(expander)
            kfn = functools.partial(_attn_kernel, vi=vi)
        else:
            kfn = functools.partial(
                lambda *a, vi: _attn_kernel(a[0], a[1], a[2], a[3], a[4],
                                            None, *a[5:], vi=vi), vi=vi)
        part = pl.pallas_call(
            kfn,
            grid=(H,),
            in_specs=in_specs,
            out_specs=[
                pl.BlockSpec((1, TQ, DH), lambda h: (h, 0, 0)),
                pl.BlockSpec((1, TQ, DH), lambda h: (h, 0, 0)),
                pl.BlockSpec((1, TQ, DH), lambda h: (h, 0, 0)),
            ],
            out_shape=[
                jax.ShapeDtypeStruct((H, TQ, DH), F32),
                jax.ShapeDtypeStruct((H, TQ, DH), F32),
                jax.ShapeDtypeStruct((H, TQ, DH), F32),
            ],
        )(*args)
        parts.append(part)
    co = jnp.concatenate([p[0] for p in parts], axis=1)
    fo = jnp.concatenate([p[1] for p in parts], axis=1)
    so = jnp.concatenate([p[2] for p in parts], axis=1)

    out = pl.pallas_call(
        _out_kernel,
        grid=(n // TQ,),
        in_specs=[
            pl.BlockSpec((H, TQ, DH), lambda i: (0, i, 0)),
            pl.BlockSpec((H, TQ, DH), lambda i: (0, i, 0)),
            pl.BlockSpec((H, TQ, DH), lambda i: (0, i, 0)),
            pl.BlockSpec((TQ, 3 * H), lambda i: (i, 0)),
            pl.BlockSpec((H, DH, d), lambda i: (0, 0, 0)),
        ],
        out_specs=pl.BlockSpec((TQ, d), lambda i: (i, 0)),
        out_shape=jax.ShapeDtypeStruct((n, d), F32),
    )(co, fo, so, gc, Wo.reshape(H, DH, d).astype(BF16))

    return out.reshape(b, n, d)


# speedup vs baseline: 1.8087x; 1.0701x over previous
import jax
import jax.numpy as jnp
from jax.experimental import pallas as pl
from jax.experimental.pallas import tpu as pltpu

B, N, D = 1, 2048, 768
H, KVH, DH = 12, 12, 64
BLK, STRIDE = 16, 8
SELBLK, NSEL = 16, 4
WIN = 64
SCALE = DH ** -0.5
NCB = (N - BLK) // STRIDE + 1
NSB = N // SELBLK
TQ = 256
BF16 = jnp.bfloat16
F32 = jnp.float32


def _bdot(a, b, dims=None):
    if dims is None:
        dims = (((a.ndim - 1,), (0,)), ((), ()))
    return jax.lax.dot_general(a.astype(BF16), b.astype(BF16), dims,
                               preferred_element_type=F32)


def _proj_kernel(x_ref, g_ref, wq_ref, wk_ref, wv_ref, wc_ref, bc_ref,
                 qb_ref, kb_ref, vb_ref, k_ref, v_ref, gq_ref):
    x = x_ref[:]
    xn = x * jax.lax.rsqrt(jnp.mean(x * x, axis=-1, keepdims=True) + 1e-6)
    xn = xn * g_ref[:]
    xnb = xn.astype(BF16)
    dd = lambda w: jax.lax.dot_general(xnb, w, (((1,), (0,)), ((), ())),
                                       preferred_element_type=F32)
    q = dd(wq_ref[:])
    k = dd(wk_ref[:])
    v = dd(wv_ref[:])
    per_head = lambda a: jnp.swapaxes(a.reshape(TQ, H, DH), 0, 1)
    qb_ref[:] = per_head((q * SCALE).astype(BF16))
    kb_ref[:] = per_head(k.astype(BF16))
    vb_ref[:] = per_head(v.astype(BF16))
    k_ref[:] = per_head(k)
    v_ref[:] = per_head(v)
    gc = jax.nn.sigmoid(dd(wc_ref[:]) + bc_ref[:])
    gq_ref[:] = jnp.swapaxes(gc, 0, 1).reshape(3, H, TQ, 1)


def _compress_kernel(kr_ref, vr_ref, kpe_ref, vpe_ref, wck_ref, wcv_ref,
                     mem_ref, ckf_ref, cvf_ref):
    k8 = kr_ref[0]
    v8 = vr_ref[0]
    half = STRIDE * DH
    kpe = kpe_ref[0]
    vpe = vpe_ref[0]
    k8s = jnp.concatenate([k8[1:], k8[:1]], axis=0)
    v8s = jnp.concatenate([v8[1:], v8[:1]], axis=0)
    kbf = jnp.concatenate([k8 + kpe[:, :half], k8s + kpe[:, half:]], axis=1)
    vbf = jnp.concatenate([v8 + vpe[:, :half], v8s + vpe[:, half:]], axis=1)
    ck = _bdot(kbf, wck_ref[0])
    cv = _bdot(vbf, wcv_ref[0])
    ckf_ref[0] = jnp.concatenate([mem_ref[0, 0], ck[:NCB]], axis=0)
    cvf_ref[0] = jnp.concatenate([mem_ref[1, 0], cv[:NCB]], axis=0)


def _attn_kernel(q_ref, k_ref, v_ref, ckf_ref, cvf_ref, gq_ref, e_ref,
                 ob_ref, *, vi):
    W = (vi + 1) * TQ
    BODY = W - TQ
    qb = q_ref[0]
    ckf = ckf_ref[0]
    cvf = cvf_ref[0]
    t = vi * TQ + jax.lax.broadcasted_iota(jnp.int32, (TQ, 1), 0)

    cs = jax.lax.dot_general(qb, ckf.astype(BF16), (((1,), (1,)), ((), ())),
                             preferred_element_type=F32)
    jc = jax.lax.broadcasted_iota(jnp.int32, (TQ, NCB + 1), 1)
    cmask = (jc == 0) | ((jc - 1) * STRIDE + BLK - 1 <= t)
    cs = jnp.where(cmask, cs, -1e30)
    cm = jnp.max(cs, axis=-1, keepdims=True)
    ce = jnp.exp(cs - cm)
    cp = ce / jnp.sum(ce, axis=-1, keepdims=True)
    c_out = _bdot(cp, cvf)

    cpr = jnp.concatenate([cp[:, 1:], cp[:, :1]], axis=1)
    impw = cp + cpr
    lane = jax.lax.broadcasted_iota(jnp.int32, (TQ, NCB + 1), 1)
    odd = (lane % 2) == 1
    impw = jnp.where(odd, impw, -1.0)
    impw = jnp.where(lane == NCB, cp[:, NCB:NCB + 1], impw)
    sels = []
    for _ in range(NSEL):
        idx = jnp.argmax(impw, axis=-1, keepdims=True).astype(jnp.int32)
        sels.append(jax.lax.shift_right_logical(idx, 1))
        impw = jnp.where(lane == idx, -2.0, impw)

    tb = t // SELBLK
    kk = k_ref[0]
    vv = v_ref[0]
    s = jax.lax.dot_general(qb, kk, (((1,), (1,)), ((), ())),
                            preferred_element_type=F32)
    rr = jax.lax.broadcasted_iota(jnp.int32, (TQ, TQ), 0)
    cc = jax.lax.broadcasted_iota(jnp.int32, (TQ, TQ), 1)
    jbq = (BODY + cc) // SELBLK
    own01 = (cc <= rr) & (jbq == tb)
    selc = ((jbq == sels[0]) | (jbq == sels[1]) | (jbq == sels[2])
            | (jbq == sels[3])) & (jbq < tb)
    chunkm = jnp.where(own01 | selc, s[:, BODY:], -1e30)
    if vi > 0:
        sblk = jax.lax.broadcasted_iota(jnp.int32, (TQ, BODY // SELBLK), 1)
        mblk = ((sblk == sels[0]) | (sblk == sels[1]) | (sblk == sels[2])
                | (sblk == sels[3]))
        m01 = jax.lax.dot_general(mblk.astype(BF16), e_ref[:],
                                  (((1,), (0,)), ((), ())),
                                  preferred_element_type=F32)
        fs = jnp.concatenate([s[:, :BODY] + (m01 * 1e30 - 1e30), chunkm],
                             axis=1)
    else:
        fs = chunkm
    fm = jnp.max(fs, axis=-1, keepdims=True)
    fe = jnp.exp(fs - fm)
    fp = (fe / jnp.sum(fe, axis=-1, keepdims=True)).astype(BF16)
    f_out = jax.lax.dot_general(fp, vv, (((1,), (0,)), ((), ())),
                                preferred_element_type=F32)

    ws0 = max(vi - 1, 0) * TQ
    WSL = W - ws0
    kw = kk[ws0:, :]
    sw = jax.lax.dot_general(qb, kw, (((1,), (1,)), ((), ())),
                             preferred_element_type=F32)
    jcw = ws0 + jax.lax.broadcasted_iota(jnp.int32, (TQ, WSL), 1)
    smask = (jcw <= t) & (t - jcw < WIN)
    wsv = jnp.where(smask, sw, -1e30)
    wm = jnp.max(wsv, axis=-1, keepdims=True)
    we = jnp.where(smask, jnp.exp(wsv - wm), 0.0)
    wp = we / jnp.sum(we, axis=-1, keepdims=True)
    s_out = _bdot(wp, vv[ws0:, :])

    ob_ref[0] = gq_ref[0, 0] * c_out + gq_ref[1, 0] * f_out \
        + gq_ref[2, 0] * s_out


def _out_kernel(cb_ref, wo_ref, o_ref):
    acc = jnp.zeros((TQ, D), F32)
    for h in range(H):
        acc = acc + _bdot(cb_ref[h], wo_ref[h])
    o_ref[:] = acc


def kernel(x, g_norm, Wq, Wk, Wv, k_pe, v_pe, Wck, Wcv, mem_kv,
           W_comb, b_comb, Wo):
    b, n, d = x.shape
    x2 = x.reshape(n, d)
    g2 = g_norm.reshape(1, d)
    b2 = b_comb.reshape(1, 3 * H)

    qb, kb, vb, k, v, gq = pl.pallas_call(
        _proj_kernel,
        grid=(n // TQ,),
        in_specs=[
            pl.BlockSpec((TQ, d), lambda i: (i, 0)),
            pl.BlockSpec((1, d), lambda i: (0, 0)),
            pl.BlockSpec((d, H * DH), lambda i: (0, 0)),
            pl.BlockSpec((d, KVH * DH), lambda i: (0, 0)),
            pl.BlockSpec((d, KVH * DH), lambda i: (0, 0)),
            pl.BlockSpec((d, 3 * H), lambda i: (0, 0)),
            pl.BlockSpec((1, 3 * H), lambda i: (0, 0)),
        ],
        out_specs=[
            pl.BlockSpec((H, TQ, DH), lambda i: (0, i, 0)),
            pl.BlockSpec((KVH, TQ, DH), lambda i: (0, i, 0)),
            pl.BlockSpec((KVH, TQ, DH), lambda i: (0, i, 0)),
            pl.BlockSpec((KVH, TQ, DH), lambda i: (0, i, 0)),
            pl.BlockSpec((KVH, TQ, DH), lambda i: (0, i, 0)),
            pl.BlockSpec((3, H, TQ, 1), lambda i: (0, 0, i, 0)),
        ],
        out_shape=[
            jax.ShapeDtypeStruct((H, n, DH), BF16),
            jax.ShapeDtypeStruct((KVH, n, DH), BF16),
            jax.ShapeDtypeStruct((KVH, n, DH), BF16),
            jax.ShapeDtypeStruct((KVH, n, DH), F32),
            jax.ShapeDtypeStruct((KVH, n, DH), F32),
            jax.ShapeDtypeStruct((3, H, n, 1), F32),
        ],
    )(x2, g2, Wq.astype(BF16), Wk.astype(BF16), Wv.astype(BF16),
      W_comb.astype(BF16), b2)

    qh, kh, vh = qb, kb, vb
    kr = k.reshape(KVH, n // STRIDE, STRIDE * DH)
    vr = v.reshape(KVH, n // STRIDE, STRIDE * DH)
    kpe2 = k_pe.reshape(KVH, 1, BLK * DH)
    vpe2 = v_pe.reshape(KVH, 1, BLK * DH)
    mem2 = mem_kv.reshape(2, KVH, 1, DH)

    ckf, cvf = pl.pallas_call(
        _compress_kernel,
        grid=(KVH,),
        in_specs=[
            pl.BlockSpec((1, n // STRIDE, STRIDE * DH), lambda h: (h, 0, 0)),
            pl.BlockSpec((1, n // STRIDE, STRIDE * DH), lambda h: (h, 0, 0)),
            pl.BlockSpec((1, 1, BLK * DH), lambda h: (h, 0, 0)),
            pl.BlockSpec((1, 1, BLK * DH), lambda h: (h, 0, 0)),
            pl.BlockSpec((1, BLK * DH, DH), lambda h: (h, 0, 0)),
            pl.BlockSpec((1, BLK * DH, DH), lambda h: (h, 0, 0)),
            pl.BlockSpec((2, 1, 1, DH), lambda h: (0, h, 0, 0)),
        ],
        out_specs=[
            pl.BlockSpec((1, NCB + 1, DH), lambda h: (h, 0, 0)),
            pl.BlockSpec((1, NCB + 1, DH), lambda h: (h, 0, 0)),
        ],
        out_shape=[
            jax.ShapeDtypeStruct((KVH, NCB + 1, DH), F32),
            jax.ShapeDtypeStruct((KVH, NCB + 1, DH), F32),
        ],
    )(kr, vr, kpe2, vpe2, Wck.astype(BF16), Wcv.astype(BF16), mem2)

    import functools
    expander = (jnp.arange(n)[None, :] // SELBLK
                == jnp.arange(NSB)[:, None]).astype(BF16)
    parts = []
    for vi in range(n // TQ):
        W = (vi + 1) * TQ
        BODY = W - TQ
        in_specs = [
            pl.BlockSpec((1, TQ, DH), lambda h, v=vi: (h, v, 0)),
            pl.BlockSpec((1, W, DH), lambda h: (h, 0, 0)),
            pl.BlockSpec((1, W, DH), lambda h: (h, 0, 0)),
            pl.BlockSpec((1, NCB + 1, DH), lambda h: (h, 0, 0)),
            pl.BlockSpec((1, NCB + 1, DH), lambda h: (h, 0, 0)),
            pl.BlockSpec((3, 1, TQ, 1), lambda h, v=vi: (0, h, v, 0)),
        ]
        args = [qh, kh, vh, ckf, cvf, gq]
        if vi > 0:
            in_specs.append(
                pl.BlockSpec((BODY // SELBLK, BODY), lambda h: (0, 0)))
            args.append(expander)
            kfn = functools.partial(_attn_kernel, vi=vi)
        else:
            kfn = functools.partial(
                lambda *a, vi: _attn_kernel(a[0], a[1], a[2], a[3], a[4],
                                            a[5], None, *a[6:], vi=vi), vi=vi)
        part = pl.pallas_call(
            kfn,
            grid=(H,),
            in_specs=in_specs,
            out_specs=pl.BlockSpec((1, TQ, DH), lambda h: (h, 0, 0)),
            out_shape=jax.ShapeDtypeStruct((H, TQ, DH), F32),
        )(*args)
        parts.append(part)
    comb = jnp.concatenate(parts, axis=1)

    out = pl.pallas_call(
        _out_kernel,
        grid=(n // TQ,),
        in_specs=[
            pl.BlockSpec((H, TQ, DH), lambda i: (0, i, 0)),
            pl.BlockSpec((H, DH, d), lambda i: (0, 0, 0)),
        ],
        out_specs=pl.BlockSpec((TQ, d), lambda i: (i, 0)),
        out_shape=jax.ShapeDtypeStruct((n, d), F32),
    )(comb, Wo.reshape(H, DH, d).astype(BF16))

    return out.reshape(b, n, d)


# speedup vs baseline: 1.8140x; 1.0029x over previous
import jax
import jax.numpy as jnp
from jax.experimental import pallas as pl
from jax.experimental.pallas import tpu as pltpu

B, N, D = 1, 2048, 768
H, KVH, DH = 12, 12, 64
BLK, STRIDE = 16, 8
SELBLK, NSEL = 16, 4
WIN = 64
SCALE = DH ** -0.5
NCB = (N - BLK) // STRIDE + 1
NSB = N // SELBLK
TQ = 256
BF16 = jnp.bfloat16
F32 = jnp.float32


def _bdot(a, b, dims=None):
    if dims is None:
        dims = (((a.ndim - 1,), (0,)), ((), ()))
    return jax.lax.dot_general(a.astype(BF16), b.astype(BF16), dims,
                               preferred_element_type=F32)


def _proj_kernel(x_ref, g_ref, wq_ref, wk_ref, wv_ref, wc_ref, bc_ref,
                 qb_ref, kb_ref, vb_ref, k_ref, v_ref, gq_ref):
    x = x_ref[:]
    xn = x * jax.lax.rsqrt(jnp.mean(x * x, axis=-1, keepdims=True) + 1e-6)
    xn = xn * g_ref[:]
    xnb = xn.astype(BF16)
    dd = lambda w: jax.lax.dot_general(xnb, w, (((1,), (0,)), ((), ())),
                                       preferred_element_type=F32)
    q = dd(wq_ref[:])
    k = dd(wk_ref[:])
    v = dd(wv_ref[:])
    per_head = lambda a: jnp.swapaxes(a.reshape(TQ, H, DH), 0, 1)
    qb_ref[:] = per_head((q * SCALE).astype(BF16))
    kb_ref[:] = per_head(k.astype(BF16))
    vb_ref[:] = per_head(v.astype(BF16))
    k_ref[:] = per_head(k)
    v_ref[:] = per_head(v)
    gc = jax.nn.sigmoid(dd(wc_ref[:]) + bc_ref[:])
    gq_ref[:] = jnp.swapaxes(gc, 0, 1).reshape(3, H, TQ, 1)


def _compress_kernel(kr_ref, vr_ref, kpe_ref, vpe_ref, wck_ref, wcv_ref,
                     mem_ref, ckf_ref, cvf_ref):
    k8 = kr_ref[0]
    v8 = vr_ref[0]
    half = STRIDE * DH
    kpe = kpe_ref[0]
    vpe = vpe_ref[0]
    k8s = jnp.concatenate([k8[1:], k8[:1]], axis=0)
    v8s = jnp.concatenate([v8[1:], v8[:1]], axis=0)
    kbf = jnp.concatenate([k8 + kpe[:, :half], k8s + kpe[:, half:]], axis=1)
    vbf = jnp.concatenate([v8 + vpe[:, :half], v8s + vpe[:, half:]], axis=1)
    ck = _bdot(kbf, wck_ref[0])
    cv = _bdot(vbf, wcv_ref[0])
    ckf_ref[0] = jnp.concatenate([mem_ref[0, 0], ck[:NCB]], axis=0)
    cvf_ref[0] = jnp.concatenate([mem_ref[1, 0], cv[:NCB]], axis=0)


def _attn_kernel(q_ref, k_ref, v_ref, ckf_ref, cvf_ref, gq_ref, e_ref,
                 ob_ref, *, vi):
    W = (vi + 1) * TQ
    BODY = W - TQ
    qb = q_ref[0]
    ckf = ckf_ref[0]
    cvf = cvf_ref[0]
    t = vi * TQ + jax.lax.broadcasted_iota(jnp.int32, (TQ, 1), 0)

    cs = jax.lax.dot_general(qb, ckf.astype(BF16), (((1,), (1,)), ((), ())),
                             preferred_element_type=F32)
    jc = jax.lax.broadcasted_iota(jnp.int32, (TQ, NCB + 1), 1)
    cmask = (jc == 0) | ((jc - 1) * STRIDE + BLK - 1 <= t)
    cs = jnp.where(cmask, cs, -1e30)
    cm = jnp.max(cs, axis=-1, keepdims=True)
    ce = jnp.exp(cs - cm)
    cp = ce / jnp.sum(ce, axis=-1, keepdims=True)
    c_out = _bdot(cp, cvf)

    cpr = jnp.concatenate([cp[:, 1:], cp[:, :1]], axis=1)
    impw = cp + cpr
    lane = jax.lax.broadcasted_iota(jnp.int32, (TQ, NCB + 1), 1)
    odd = (lane % 2) == 1
    impw = jnp.where(odd, impw, -1.0)
    impw = jnp.where(lane == NCB, cp[:, NCB:NCB + 1], impw)
    sels = []
    for _ in range(NSEL):
        idx = jnp.argmax(impw, axis=-1, keepdims=True).astype(jnp.int32)
        sels.append(jax.lax.shift_right_logical(idx, 1))
        impw = jnp.where(lane == idx, -2.0, impw)

    tb = t // SELBLK
    kk = k_ref[0]
    vv = v_ref[0]
    s = jax.lax.dot_general(qb, kk, (((1,), (1,)), ((), ())),
                            preferred_element_type=F32)
    rr = jax.lax.broadcasted_iota(jnp.int32, (TQ, TQ), 0)
    cc = jax.lax.broadcasted_iota(jnp.int32, (TQ, TQ), 1)
    jbq = (BODY + cc) // SELBLK
    own01 = (cc <= rr) & (jbq == tb)
    selc = ((jbq == sels[0]) | (jbq == sels[1]) | (jbq == sels[2])
            | (jbq == sels[3])) & (jbq < tb)
    chunkm = jnp.where(own01 | selc, s[:, BODY:], -1e30)
    if vi > 0:
        sblk = jax.lax.broadcasted_iota(jnp.int32, (TQ, BODY // SELBLK), 1)
        mblk = ((sblk == sels[0]) | (sblk == sels[1]) | (sblk == sels[2])
                | (sblk == sels[3]))
        m01 = jax.lax.dot_general(mblk.astype(BF16), e_ref[:],
                                  (((1,), (0,)), ((), ())),
                                  preferred_element_type=F32)
        fs = jnp.concatenate([s[:, :BODY] + (m01 * 1e30 - 1e30), chunkm],
                             axis=1)
    else:
        fs = chunkm
    fm = jnp.max(fs, axis=-1, keepdims=True)
    fe = jnp.exp(fs - fm)
    fp = (fe / jnp.sum(fe, axis=-1, keepdims=True)).astype(BF16)
    f_out = jax.lax.dot_general(fp, vv, (((1,), (0,)), ((), ())),
                                preferred_element_type=F32)

    ws0 = max(vi - 1, 0) * TQ
    WSL = W - ws0
    sw = s[:, ws0:]
    jcw = ws0 + jax.lax.broadcasted_iota(jnp.int32, (TQ, WSL), 1)
    smask = (jcw <= t) & (t - jcw < WIN)
    wsv = jnp.where(smask, sw, -1e30)
    wm = jnp.max(wsv, axis=-1, keepdims=True)
    we = jnp.where(smask, jnp.exp(wsv - wm), 0.0)
    wp = we / jnp.sum(we, axis=-1, keepdims=True)
    s_out = _bdot(wp, vv[ws0:, :])

    ob_ref[0] = gq_ref[0, 0] * c_out + gq_ref[1, 0] * f_out \
        + gq_ref[2, 0] * s_out


def _out_kernel(cb_ref, wo_ref, o_ref):
    acc = jnp.zeros((TQ, D), F32)
    for h in range(H):
        acc = acc + _bdot(cb_ref[h], wo_ref[h])
    o_ref[:] = acc


def kernel(x, g_norm, Wq, Wk, Wv, k_pe, v_pe, Wck, Wcv, mem_kv,
           W_comb, b_comb, Wo):
    b, n, d = x.shape
    x2 = x.reshape(n, d)
    g2 = g_norm.reshape(1, d)
    b2 = b_comb.reshape(1, 3 * H)

    qb, kb, vb, k, v, gq = pl.pallas_call(
        _proj_kernel,
        grid=(n // TQ,),
        in_specs=[
            pl.BlockSpec((TQ, d), lambda i: (i, 0)),
            pl.BlockSpec((1, d), lambda i: (0, 0)),
            pl.BlockSpec((d, H * DH), lambda i: (0, 0)),
            pl.BlockSpec((d, KVH * DH), lambda i: (0, 0)),
            pl.BlockSpec((d, KVH * DH), lambda i: (0, 0)),
            pl.BlockSpec((d, 3 * H), lambda i: (0, 0)),
            pl.BlockSpec((1, 3 * H), lambda i: (0, 0)),
        ],
        out_specs=[
            pl.BlockSpec((H, TQ, DH), lambda i: (0, i, 0)),
            pl.BlockSpec((KVH, TQ, DH), lambda i: (0, i, 0)),
            pl.BlockSpec((KVH, TQ, DH), lambda i: (0, i, 0)),
            pl.BlockSpec((KVH, TQ, DH), lambda i: (0, i, 0)),
            pl.BlockSpec((KVH, TQ, DH), lambda i: (0, i, 0)),
            pl.BlockSpec((3, H, TQ, 1), lambda i: (0, 0, i, 0)),
        ],
        out_shape=[
            jax.ShapeDtypeStruct((H, n, DH), BF16),
            jax.ShapeDtypeStruct((KVH, n, DH), BF16),
            jax.ShapeDtypeStruct((KVH, n, DH), BF16),
            jax.ShapeDtypeStruct((KVH, n, DH), F32),
            jax.ShapeDtypeStruct((KVH, n, DH), F32),
            jax.ShapeDtypeStruct((3, H, n, 1), F32),
        ],
    )(x2, g2, Wq.astype(BF16), Wk.astype(BF16), Wv.astype(BF16),
      W_comb.astype(BF16), b2)

    qh, kh, vh = qb, kb, vb
    kr = k.reshape(KVH, n // STRIDE, STRIDE * DH)
    vr = v.reshape(KVH, n // STRIDE, STRIDE * DH)
    kpe2 = k_pe.reshape(KVH, 1, BLK * DH)
    vpe2 = v_pe.reshape(KVH, 1, BLK * DH)
    mem2 = mem_kv.reshape(2, KVH, 1, DH)

    ckf, cvf = pl.pallas_call(
        _compress_kernel,
        grid=(KVH,),
        in_specs=[
            pl.BlockSpec((1, n // STRIDE, STRIDE * DH), lambda h: (h, 0, 0)),
            pl.BlockSpec((1, n // STRIDE, STRIDE * DH), lambda h: (h, 0, 0)),
            pl.BlockSpec((1, 1, BLK * DH), lambda h: (h, 0, 0)),
            pl.BlockSpec((1, 1, BLK * DH), lambda h: (h, 0, 0)),
            pl.BlockSpec((1, BLK * DH, DH), lambda h: (h, 0, 0)),
            pl.BlockSpec((1, BLK * DH, DH), lambda h: (h, 0, 0)),
            pl.BlockSpec((2, 1, 1, DH), lambda h: (0, h, 0, 0)),
        ],
        out_specs=[
            pl.BlockSpec((1, NCB + 1, DH), lambda h: (h, 0, 0)),
            pl.BlockSpec((1, NCB + 1, DH), lambda h: (h, 0, 0)),
        ],
        out_shape=[
            jax.ShapeDtypeStruct((KVH, NCB + 1, DH), F32),
            jax.ShapeDtypeStruct((KVH, NCB + 1, DH), F32),
        ],
    )(kr, vr, kpe2, vpe2, Wck.astype(BF16), Wcv.astype(BF16), mem2)

    import functools
    expander = (jnp.arange(n)[None, :] // SELBLK
                == jnp.arange(NSB)[:, None]).astype(BF16)
    parts = []
    for vi in range(n // TQ):
        W = (vi + 1) * TQ
        BODY = W - TQ
        in_specs = [
            pl.BlockSpec((1, TQ, DH), lambda h, v=vi: (h, v, 0)),
            pl.BlockSpec((1, W, DH), lambda h: (h, 0, 0)),
            pl.BlockSpec((1, W, DH), lambda h: (h, 0, 0)),
            pl.BlockSpec((1, NCB + 1, DH), lambda h: (h, 0, 0)),
            pl.BlockSpec((1, NCB + 1, DH), lambda h: (h, 0, 0)),
            pl.BlockSpec((3, 1, TQ, 1), lambda h, v=vi: (0, h, v, 0)),
        ]
        args = [qh, kh, vh, ckf, cvf, gq]
        if vi > 0:
            in_specs.append(
                pl.BlockSpec((BODY // SELBLK, BODY), lambda h: (0, 0)))
            args.append(expander)
            kfn = functools.partial(_attn_kernel, vi=vi)
        else:
            kfn = functools.partial(
                lambda *a, vi: _attn_kernel(a[0], a[1], a[2], a[3], a[4],
                                            a[5], None, *a[6:], vi=vi), vi=vi)
        part = pl.pallas_call(
            kfn,
            grid=(H,),
            in_specs=in_specs,
            out_specs=pl.BlockSpec((1, TQ, DH), lambda h: (h, 0, 0)),
            out_shape=jax.ShapeDtypeStruct((H, TQ, DH), F32),
        )(*args)
        parts.append(part)
    comb = jnp.concatenate(parts, axis=1)

    out = pl.pallas_call(
        _out_kernel,
        grid=(n // TQ,),
        in_specs=[
            pl.BlockSpec((H, TQ, DH), lambda i: (0, i, 0)),
            pl.BlockSpec((H, DH, d), lambda i: (0, 0, 0)),
        ],
        out_specs=pl.BlockSpec((TQ, d), lambda i: (i, 0)),
        out_shape=jax.ShapeDtypeStruct((n, d), F32),
    )(comb, Wo.reshape(H, DH, d).astype(BF16))

    return out.reshape(b, n, d)


# speedup vs baseline: 1.8248x; 1.0059x over previous
import jax
import jax.numpy as jnp
from jax.experimental import pallas as pl
from jax.experimental.pallas import tpu as pltpu

B, N, D = 1, 2048, 768
H, KVH, DH = 12, 12, 64
BLK, STRIDE = 16, 8
SELBLK, NSEL = 16, 4
WIN = 64
SCALE = DH ** -0.5
NCB = (N - BLK) // STRIDE + 1
NSB = N // SELBLK
TQ = 256
BF16 = jnp.bfloat16
F32 = jnp.float32


def _bdot(a, b, dims=None):
    if dims is None:
        dims = (((a.ndim - 1,), (0,)), ((), ()))
    return jax.lax.dot_general(a.astype(BF16), b.astype(BF16), dims,
                               preferred_element_type=F32)


def _proj_kernel(x_ref, g_ref, wq_ref, wk_ref, wv_ref, wc_ref, bc_ref,
                 qb_ref, kb_ref, vb_ref, k_ref, v_ref, gq_ref):
    x = x_ref[:]
    xn = x * jax.lax.rsqrt(jnp.mean(x * x, axis=-1, keepdims=True) + 1e-6)
    xn = xn * g_ref[:]
    xnb = xn.astype(BF16)
    dd = lambda w: jax.lax.dot_general(xnb, w, (((1,), (0,)), ((), ())),
                                       preferred_element_type=F32)
    q = dd(wq_ref[:])
    k = dd(wk_ref[:])
    v = dd(wv_ref[:])
    per_head = lambda a: jnp.swapaxes(a.reshape(TQ, H, DH), 0, 1)
    qb_ref[:] = per_head((q * SCALE).astype(BF16))
    kb_ref[:] = per_head(k.astype(BF16))
    vb_ref[:] = per_head(v.astype(BF16))
    k_ref[:] = per_head(k)
    v_ref[:] = per_head(v)
    gc = jax.nn.sigmoid(dd(wc_ref[:]) + bc_ref[:])
    gq_ref[:] = jnp.swapaxes(gc, 0, 1).reshape(3, H, TQ, 1)


def _compress_kernel(kr_ref, vr_ref, kpe_ref, vpe_ref, wck_ref, wcv_ref,
                     mem_ref, ckf_ref, cvf_ref):
    k8 = kr_ref[0]
    v8 = vr_ref[0]
    half = STRIDE * DH
    kpe = kpe_ref[0]
    vpe = vpe_ref[0]
    k8s = jnp.concatenate([k8[1:], k8[:1]], axis=0)
    v8s = jnp.concatenate([v8[1:], v8[:1]], axis=0)
    kbf = jnp.concatenate([k8 + kpe[:, :half], k8s + kpe[:, half:]], axis=1)
    vbf = jnp.concatenate([v8 + vpe[:, :half], v8s + vpe[:, half:]], axis=1)
    ck = _bdot(kbf, wck_ref[0])
    cv = _bdot(vbf, wcv_ref[0])
    ckf_ref[0] = jnp.concatenate([mem_ref[0, 0], ck[:NCB]], axis=0)
    cvf_ref[0] = jnp.concatenate([mem_ref[1, 0], cv[:NCB]], axis=0)


def _attn_kernel(q_ref, k_ref, v_ref, ckf_ref, cvf_ref, gq_ref, e_ref,
                 ob_ref, *, vi):
    W = (vi + 1) * TQ
    BODY = W - TQ
    qb = q_ref[0]
    ckf = ckf_ref[0]
    cvf = cvf_ref[0]
    t = vi * TQ + jax.lax.broadcasted_iota(jnp.int32, (TQ, 1), 0)

    cs = jax.lax.dot_general(qb, ckf.astype(BF16), (((1,), (1,)), ((), ())),
                             preferred_element_type=F32)
    jc = jax.lax.broadcasted_iota(jnp.int32, (TQ, NCB + 1), 1)
    cmask = (jc == 0) | ((jc - 1) * STRIDE + BLK - 1 <= t)
    cs = jnp.where(cmask, cs, -1e30)
    cm = jnp.max(cs, axis=-1, keepdims=True)
    ce = jnp.exp(cs - cm)
    cp = ce / jnp.sum(ce, axis=-1, keepdims=True)
    c_out = _bdot(cp, cvf)

    cpr = jnp.concatenate([cp[:, 1:], cp[:, :1]], axis=1)
    impw = cp + cpr
    lane = jax.lax.broadcasted_iota(jnp.int32, (TQ, NCB + 1), 1)
    odd = (lane % 2) == 1
    impw = jnp.where(odd, impw, -1.0)
    impw = jnp.where(lane == NCB, cp[:, NCB:NCB + 1], impw)
    sels = []
    for _ in range(NSEL):
        idx = jnp.argmax(impw, axis=-1, keepdims=True).astype(jnp.int32)
        sels.append(jax.lax.shift_right_logical(idx, 1))
        impw = jnp.where(lane == idx, -2.0, impw)

    tb = t // SELBLK
    kk = k_ref[0]
    vv = v_ref[0]
    s = jax.lax.dot_general(qb, kk, (((1,), (1,)), ((), ())),
                            preferred_element_type=F32)
    rr = jax.lax.broadcasted_iota(jnp.int32, (TQ, TQ), 0)
    cc = jax.lax.broadcasted_iota(jnp.int32, (TQ, TQ), 1)
    jbq = (BODY + cc) // SELBLK
    own01 = (cc <= rr) & (jbq == tb)
    selc = ((jbq == sels[0]) | (jbq == sels[1]) | (jbq == sels[2])
            | (jbq == sels[3])) & (jbq < tb)
    chunkm = jnp.where(own01 | selc, s[:, BODY:], -1e30)
    if vi > 0:
        sblk = jax.lax.broadcasted_iota(jnp.int32, (TQ, BODY // SELBLK), 1)
        mblk = ((sblk == sels[0]) | (sblk == sels[1]) | (sblk == sels[2])
                | (sblk == sels[3]))
        m01 = jax.lax.dot_general(mblk.astype(BF16), e_ref[:],
                                  (((1,), (0,)), ((), ())),
                                  preferred_element_type=F32)
        fs = jnp.concatenate([s[:, :BODY] + (m01 * 1e30 - 1e30), chunkm],
                             axis=1)
    else:
        fs = chunkm
    fm = jnp.max(fs, axis=-1, keepdims=True)
    fe = jnp.exp(fs - fm)
    fp = (fe / jnp.sum(fe, axis=-1, keepdims=True)).astype(BF16)
    f_out = jax.lax.dot_general(fp, vv, (((1,), (0,)), ((), ())),
                                preferred_element_type=F32)

    ws0 = max(vi - 1, 0) * TQ
    WSL = W - ws0
    sw = s[:, ws0:]
    jcw = ws0 + jax.lax.broadcasted_iota(jnp.int32, (TQ, WSL), 1)
    smask = (jcw <= t) & (t - jcw < WIN)
    wsv = jnp.where(smask, sw, -1e30)
    wm = jnp.max(wsv, axis=-1, keepdims=True)
    we = jnp.where(smask, jnp.exp(wsv - wm), 0.0)
    wp = we / jnp.sum(we, axis=-1, keepdims=True)
    s_out = _bdot(wp, vv[ws0:, :])

    ob_ref[0] = gq_ref[0, 0] * c_out + gq_ref[1, 0] * f_out \
        + gq_ref[2, 0] * s_out


def _out_kernel(cb_ref, wo_ref, o_ref):
    comb = jnp.swapaxes(cb_ref[:], 0, 1).reshape(TQ, H * DH)
    o_ref[:] = _bdot(comb, wo_ref[:])


def kernel(x, g_norm, Wq, Wk, Wv, k_pe, v_pe, Wck, Wcv, mem_kv,
           W_comb, b_comb, Wo):
    b, n, d = x.shape
    x2 = x.reshape(n, d)
    g2 = g_norm.reshape(1, d)
    b2 = b_comb.reshape(1, 3 * H)

    qb, kb, vb, k, v, gq = pl.pallas_call(
        _proj_kernel,
        grid=(n // TQ,),
        in_specs=[
            pl.BlockSpec((TQ, d), lambda i: (i, 0)),
            pl.BlockSpec((1, d), lambda i: (0, 0)),
            pl.BlockSpec((d, H * DH), lambda i: (0, 0)),
            pl.BlockSpec((d, KVH * DH), lambda i: (0, 0)),
            pl.BlockSpec((d, KVH * DH), lambda i: (0, 0)),
            pl.BlockSpec((d, 3 * H), lambda i: (0, 0)),
            pl.BlockSpec((1, 3 * H), lambda i: (0, 0)),
        ],
        out_specs=[
            pl.BlockSpec((H, TQ, DH), lambda i: (0, i, 0)),
            pl.BlockSpec((KVH, TQ, DH), lambda i: (0, i, 0)),
            pl.BlockSpec((KVH, TQ, DH), lambda i: (0, i, 0)),
            pl.BlockSpec((KVH, TQ, DH), lambda i: (0, i, 0)),
            pl.BlockSpec((KVH, TQ, DH), lambda i: (0, i, 0)),
            pl.BlockSpec((3, H, TQ, 1), lambda i: (0, 0, i, 0)),
        ],
        out_shape=[
            jax.ShapeDtypeStruct((H, n, DH), BF16),
            jax.ShapeDtypeStruct((KVH, n, DH), BF16),
            jax.ShapeDtypeStruct((KVH, n, DH), BF16),
            jax.ShapeDtypeStruct((KVH, n, DH), F32),
            jax.ShapeDtypeStruct((KVH, n, DH), F32),
            jax.ShapeDtypeStruct((3, H, n, 1), F32),
        ],
    )(x2, g2, Wq.astype(BF16), Wk.astype(BF16), Wv.astype(BF16),
      W_comb.astype(BF16), b2)

    qh, kh, vh = qb, kb, vb
    kr = k.reshape(KVH, n // STRIDE, STRIDE * DH)
    vr = v.reshape(KVH, n // STRIDE, STRIDE * DH)
    kpe2 = k_pe.reshape(KVH, 1, BLK * DH)
    vpe2 = v_pe.reshape(KVH, 1, BLK * DH)
    mem2 = mem_kv.reshape(2, KVH, 1, DH)

    ckf, cvf = pl.pallas_call(
        _compress_kernel,
        grid=(KVH,),
        in_specs=[
            pl.BlockSpec((1, n // STRIDE, STRIDE * DH), lambda h: (h, 0, 0)),
            pl.BlockSpec((1, n // STRIDE, STRIDE * DH), lambda h: (h, 0, 0)),
            pl.BlockSpec((1, 1, BLK * DH), lambda h: (h, 0, 0)),
            pl.BlockSpec((1, 1, BLK * DH), lambda h: (h, 0, 0)),
            pl.BlockSpec((1, BLK * DH, DH), lambda h: (h, 0, 0)),
            pl.BlockSpec((1, BLK * DH, DH), lambda h: (h, 0, 0)),
            pl.BlockSpec((2, 1, 1, DH), lambda h: (0, h, 0, 0)),
        ],
        out_specs=[
            pl.BlockSpec((1, NCB + 1, DH), lambda h: (h, 0, 0)),
            pl.BlockSpec((1, NCB + 1, DH), lambda h: (h, 0, 0)),
        ],
        out_shape=[
            jax.ShapeDtypeStruct((KVH, NCB + 1, DH), F32),
            jax.ShapeDtypeStruct((KVH, NCB + 1, DH), F32),
        ],
    )(kr, vr, kpe2, vpe2, Wck.astype(BF16), Wcv.astype(BF16), mem2)

    import functools
    expander = (jnp.arange(n)[None, :] // SELBLK
                == jnp.arange(NSB)[:, None]).astype(BF16)
    parts = []
    for vi in range(n // TQ):
        W = (vi + 1) * TQ
        BODY = W - TQ
        in_specs = [
            pl.BlockSpec((1, TQ, DH), lambda h, v=vi: (h, v, 0)),
            pl.BlockSpec((1, W, DH), lambda h: (h, 0, 0)),
            pl.BlockSpec((1, W, DH), lambda h: (h, 0, 0)),
            pl.BlockSpec((1, NCB + 1, DH), lambda h: (h, 0, 0)),
            pl.BlockSpec((1, NCB + 1, DH), lambda h: (h, 0, 0)),
            pl.BlockSpec((3, 1, TQ, 1), lambda h, v=vi: (0, h, v, 0)),
        ]
        args = [qh, kh, vh, ckf, cvf, gq]
        if vi > 0:
            in_specs.append(
                pl.BlockSpec((BODY // SELBLK, BODY), lambda h: (0, 0)))
            args.append(expander)
            kfn = functools.partial(_attn_kernel, vi=vi)
        else:
            kfn = functools.partial(
                lambda *a, vi: _attn_kernel(a[0], a[1], a[2], a[3], a[4],
                                            a[5], None, *a[6:], vi=vi), vi=vi)
        part = pl.pallas_call(
            kfn,
            grid=(H,),
            in_specs=in_specs,
            out_specs=pl.BlockSpec((1, TQ, DH), lambda h: (h, 0, 0)),
            out_shape=jax.ShapeDtypeStruct((H, TQ, DH), F32),
        )(*args)
        parts.append(part)
    comb = jnp.concatenate(parts, axis=1)

    out = pl.pallas_call(
        _out_kernel,
        grid=(n // TQ,),
        in_specs=[
            pl.BlockSpec((H, TQ, DH), lambda i: (0, i, 0)),
            pl.BlockSpec((H * DH, d), lambda i: (0, 0)),
        ],
        out_specs=pl.BlockSpec((TQ, d), lambda i: (i, 0)),
        out_shape=jax.ShapeDtypeStruct((n, d), F32),
    )(comb, Wo.astype(BF16))

    return out.reshape(b, n, d)
